# baseline (speedup 1.0000x reference)
import functools
import jax
import jax.numpy as jnp
from jax import lax
from jax.experimental import pallas as pl
from jax.experimental.pallas import tpu as pltpu
from jax.experimental.pallas import tpu_sc as plsc

B, S = 4, 4096
WS, WH = 64, 128
V, TH = 1000000, 128

NC, NS = 2, 16
NW = NC * NS
N = B * S
BPW = N // NW
CH = 64
NCH = BPW // CH
WPIECE = WS * WH // (NW // B)


def _body(ids_hbm, ws_hbm, table_hbm, ws_out, emb_out, idx_v, rows_v, ws_v,
          *sems):
    gsems, osem = sems[:NCH], sems[NCH]
    wid = lax.axis_index("s") * NC + lax.axis_index("c")

    pltpu.sync_copy(ids_hbm.at[wid], idx_v)

    gathers = []
    for j in range(NCH):
        gathers.append(
            pltpu.async_copy(
                table_hbm.at[idx_v.at[j]],
                rows_v.at[pl.ds(j * CH, CH)],
                gsems[j],
            )
        )

    piece = lax.rem(wid, NW // B) * WPIECE
    batch = lax.div(wid, NW // B)
    pltpu.sync_copy(ws_hbm.at[pl.ds(piece, WPIECE)], ws_v)
    pltpu.sync_copy(ws_v, ws_out.at[batch, pl.ds(piece, WPIECE)])

    out_copies = []
    for j in range(NCH):
        gathers[j].wait()
        out_copies.append(
            pltpu.async_copy(
                rows_v.at[pl.ds(j * CH, CH)],
                emb_out.at[wid, pl.ds(j * CH, CH)],
                osem,
            )
        )
    for c in out_copies:
        c.wait()


@jax.jit
def _run(ids, ws_flat, table):
    kern = pl.kernel(
        _body,
        out_type=(
            jax.ShapeDtypeStruct((B, WS * WH), jnp.float32),
            jax.ShapeDtypeStruct((NW, BPW, TH), jnp.float32),
        ),
        mesh=plsc.VectorSubcoreMesh(core_axis_name="c", subcore_axis_name="s"),
        scratch_types=[
            pltpu.VMEM((NCH, CH), jnp.int32),
            pltpu.VMEM((BPW, TH), jnp.float32),
            pltpu.VMEM((WPIECE,), jnp.float32),
        ] + [pltpu.SemaphoreType.DMA] * (NCH + 1),
    )
    return kern(ids, ws_flat, table)


def kernel(input_ids, attention_mask, init_workspace, word_table):
    ids = input_ids.reshape(NW, NCH, CH)
    ws_flat = init_workspace.reshape(WS * WH)
    ws_out, emb = _run(ids, ws_flat, word_table)
    workspace = ws_out.reshape(B, WS, WH)
    embeddings = emb.reshape(B, S, TH)
    return (workspace, embeddings)


# speedup vs baseline: 1.0668x; 1.0668x over previous
import jax
import jax.numpy as jnp
from jax import lax
from jax.experimental import pallas as pl
from jax.experimental.pallas import tpu as pltpu
from jax.experimental.pallas import tpu_sc as plsc

B, S = 4, 4096
WS, WH = 64, 128
V, TH = 1000000, 128

NC, NS = 2, 16
NW = NC * NS
WPB = NW // B
BPW = S // WPB
CH = 128
NCH = BPW // CH
WROWS = WS // WPB


def _body(ids_hbm, ws_hbm, table_hbm, ws_out, emb_out, idx_v, rows_v, ws_v,
          *sems):
    gsems, osem = sems[:NCH], sems[NCH]
    wid = lax.axis_index("s") * NC + lax.axis_index("c")
    batch = lax.div(wid, WPB)
    stripe = lax.rem(wid, WPB)

    pltpu.sync_copy(ids_hbm.at[batch, stripe], idx_v)

    gathers = []
    for j in range(NCH):
        gathers.append(
            pltpu.async_copy(
                table_hbm.at[idx_v.at[j]],
                rows_v.at[pl.ds(j * CH, CH)],
                gsems[j],
            )
        )

    tok0 = stripe * BPW
    out_copies = []
    for j in range(NCH):
        gathers[j].wait()
        out_copies.append(
            pltpu.async_copy(
                rows_v.at[pl.ds(j * CH, CH)],
                emb_out.at[batch, pl.ds(tok0 + j * CH, CH)],
                osem,
            )
        )

    row0 = stripe * WROWS
    pltpu.sync_copy(ws_hbm.at[pl.ds(row0, WROWS)], ws_v)
    pltpu.sync_copy(ws_v, ws_out.at[batch, pl.ds(row0, WROWS)])

    for c in out_copies:
        c.wait()


@jax.jit
def _run(ids, ws, table):
    kern = pl.kernel(
        _body,
        out_type=(
            jax.ShapeDtypeStruct((B, WS, WH), jnp.float32),
            jax.ShapeDtypeStruct((B, S, TH), jnp.float32),
        ),
        mesh=plsc.VectorSubcoreMesh(core_axis_name="c", subcore_axis_name="s"),
        scratch_types=[
            pltpu.VMEM((NCH, CH), jnp.int32),
            pltpu.VMEM((BPW, TH), jnp.float32),
            pltpu.VMEM((WROWS, WH), jnp.float32),
        ] + [pltpu.SemaphoreType.DMA] * (NCH + 1),
    )
    return kern(ids, ws, table)


def kernel(input_ids, attention_mask, init_workspace, word_table):
    ids = input_ids.reshape(B, WPB, NCH, CH)
    ws = init_workspace.reshape(WS, WH)
    workspace, embeddings = _run(ids, ws, word_table)
    return (workspace, embeddings)
